# Initial kernel scaffold; baseline (speedup 1.0000x reference)
import jax
import jax.numpy as jnp
from jax.experimental import pallas as pl


def kernel(inputBatch, Wg, W1, W2):
    raise NotImplementedError("write your pallas kernel here")


# speedup vs baseline: 9.8078x; 9.8078x over previous
import functools

import jax
import jax.numpy as jnp
from jax import lax
from jax.experimental import pallas as pl
from jax.experimental.pallas import tpu as pltpu
from jax.experimental.pallas import tpu_sc as plsc

B = 4
S = 2048
D = 768
H = 768
E = 64
K = 2
N = B * S
NK = N * K

TB = 512
NA = N // TB


def _router_body(x_ref, wg_ref, xg0_ref, xg1_ref, ii_ref):
    x = x_ref[:]
    logits = jnp.dot(x, wg_ref[:], preferred_element_type=jnp.float32)
    m = jnp.max(logits, axis=1, keepdims=True)
    ex = jnp.exp(logits - m)
    probs = ex / jnp.sum(ex, axis=1, keepdims=True)
    lane = lax.broadcasted_iota(jnp.int32, (TB, E), 1)
    p0 = jnp.max(probs, axis=1, keepdims=True)
    i0 = jnp.min(jnp.where(probs == p0, lane, E), axis=1, keepdims=True)
    probs2 = jnp.where(lane == i0, -jnp.inf, probs)
    p1 = jnp.max(probs2, axis=1, keepdims=True)
    i1 = jnp.min(jnp.where(probs2 == p1, lane, E), axis=1, keepdims=True)
    xg0_ref[:] = x * p0
    xg1_ref[:] = x * p1
    ii_ref[:] = jnp.concatenate([i0, i1], axis=1)


def _router(xf, Wg):
    return pl.pallas_call(
        _router_body,
        grid=(NA,),
        in_specs=[
            pl.BlockSpec((TB, D), lambda i: (i, 0)),
            pl.BlockSpec((D, E), lambda i: (0, 0)),
        ],
        out_specs=[
            pl.BlockSpec((TB, D), lambda i: (i, 0)),
            pl.BlockSpec((TB, D), lambda i: (i, 0)),
            pl.BlockSpec((TB, K), lambda i: (i, 0)),
        ],
        out_shape=[
            jax.ShapeDtypeStruct((N, D), jnp.float32),
            jax.ShapeDtypeStruct((N, D), jnp.float32),
            jax.ShapeDtypeStruct((N, K), jnp.int32),
        ],
    )(xf, Wg)


GC = 256
GR = NK // GC


def _sortpos_body(ids_ref, pos_ref, hist_ref, acc_ref):
    p = pl.program_id(0)
    r = pl.program_id(1)
    ids = ids_ref[0]
    eidx = lax.broadcasted_iota(jnp.int32, (E, GC), 0)
    onehot = (eidx == ids).astype(jnp.float32)

    @pl.when((p == 0) & (r == 0))
    def _():
        acc_ref[:] = jnp.zeros((E, GC), jnp.float32)

    @pl.when((p == 1) & (r == 0))
    def _():
        totals = acc_ref[:]
        hist_ref[:] = totals.astype(jnp.int32)
        er0 = lax.broadcasted_iota(jnp.int32, (E, E), 0)
        er1 = lax.broadcasted_iota(jnp.int32, (E, E), 1)
        sle = (er1 < er0).astype(jnp.float32)
        acc_ref[:] = jnp.dot(sle, totals, preferred_element_type=jnp.float32,
                             precision=lax.Precision.HIGHEST)

    @pl.when(p == 1)
    def _():
        c0 = lax.broadcasted_iota(jnp.int32, (GC, GC), 0)
        c1 = lax.broadcasted_iota(jnp.int32, (GC, GC), 1)
        su = (c0 < c1).astype(jnp.float32)
        rnk = jnp.dot(onehot, su, preferred_element_type=jnp.float32,
                      precision=lax.Precision.HIGHEST)
        rank = jnp.sum(rnk * onehot, axis=0, keepdims=True)
        base = jnp.sum(acc_ref[:] * onehot, axis=0, keepdims=True)
        pos_ref[0] = (base + rank).astype(jnp.int32)

    ones = jnp.ones((GC, GC), jnp.float32)
    acc_ref[:] = acc_ref[:] + jnp.dot(onehot, ones,
                                      preferred_element_type=jnp.float32,
                                      precision=lax.Precision.HIGHEST)


def _sortpos(ids3):
    return pl.pallas_call(
        _sortpos_body,
        grid=(2, GR),
        in_specs=[pl.BlockSpec((1, 1, GC), lambda p, r: (r, 0, 0))],
        out_specs=[
            pl.BlockSpec((1, 1, GC),
                         lambda p, r: (jnp.where(p == 1, r, 0), 0, 0)),
            pl.BlockSpec((E, GC), lambda p, r: (0, 0)),
        ],
        out_shape=[
            jax.ShapeDtypeStruct((GR, 1, GC), jnp.int32),
            jax.ShapeDtypeStruct((E, GC), jnp.int32),
        ],
        scratch_shapes=[pltpu.VMEM((E, GC), jnp.float32)],
    )(ids3)


TM = 256
NT = NK // TM
G = NT + E - 1


def _ffn_body(tid_ref, eid_ref, first_ref, valid_ref, off_ref,
              x_ref, w1_ref, w2_ref, out_ref):
    g = pl.program_id(0)

    @pl.when(valid_ref[g] == 1)
    def _():
        e = eid_ref[g]
        t = tid_ref[g]
        lo = off_ref[e]
        hi = off_ref[e + 1]
        rows = t * TM + lax.broadcasted_iota(jnp.int32, (TM, 1), 0)
        msk = (rows >= lo) & (rows < hi)
        x = jnp.where(msk, x_ref[:], 0.0)
        h = jnp.maximum(
            jnp.dot(x, w1_ref[0], preferred_element_type=jnp.float32), 0.0)
        part = jnp.dot(h, w2_ref[0], preferred_element_type=jnp.float32)

        @pl.when(first_ref[g] == 1)
        def _():
            out_ref[:] = part

        @pl.when(first_ref[g] == 0)
        def _():
            out_ref[:] = out_ref[:] + part


def _ffn(tid, eid, first, valid, offsets, Xs, W1, W2):
    grid_spec = pltpu.PrefetchScalarGridSpec(
        num_scalar_prefetch=5,
        grid=(G,),
        in_specs=[
            pl.BlockSpec((TM, D), lambda g, t, e, f, v, o: (t[g], 0)),
            pl.BlockSpec((1, D, H), lambda g, t, e, f, v, o: (e[g], 0, 0)),
            pl.BlockSpec((1, H, D), lambda g, t, e, f, v, o: (e[g], 0, 0)),
        ],
        out_specs=pl.BlockSpec((TM, D), lambda g, t, e, f, v, o: (t[g], 0)),
    )
    return pl.pallas_call(
        _ffn_body,
        grid_spec=grid_spec,
        out_shape=jax.ShapeDtypeStruct((NK, D), jnp.float32),
    )(tid, eid, first, valid, offsets, Xs, W1, W2)


_NC, _NS = 2, 16
_NW = _NC * _NS
CH = 64
NCH = (N // _NW) // CH


def _sc_mesh():
    return plsc.VectorSubcoreMesh(core_axis_name="c", subcore_axis_name="s",
                                  num_cores=_NC, num_subcores=_NS)


def _scatter(xg0, xg1, pe3, po3):
    @functools.partial(
        pl.kernel,
        out_type=jax.ShapeDtypeStruct((NK, D), jnp.float32),
        mesh=_sc_mesh(),
        scratch_types=[
            pltpu.VMEM((CH, D), jnp.float32),
            pltpu.VMEM((NCH, CH), jnp.int32),
            pltpu.VMEM((NCH, CH), jnp.int32),
            pltpu.SemaphoreType.DMA,
        ],
    )
    def k(xg0_hbm, xg1_hbm, pe_hbm, po_hbm, out_hbm, rowbuf, idxe, idxo, sem):
        w = lax.axis_index("s") * _NC + lax.axis_index("c")
        base = w * (N // _NW)
        pltpu.sync_copy(pe_hbm.at[w], idxe)
        pltpu.sync_copy(po_hbm.at[w], idxo)
        for j in range(NCH):
            pltpu.sync_copy(xg0_hbm.at[pl.ds(base + j * CH, CH)], rowbuf)
            pltpu.async_copy(rowbuf, out_hbm.at[idxe.at[j]], sem).wait()
            pltpu.sync_copy(xg1_hbm.at[pl.ds(base + j * CH, CH)], rowbuf)
            pltpu.async_copy(rowbuf, out_hbm.at[idxo.at[j]], sem).wait()

    return k(xg0, xg1, pe3, po3)


def _combine(xout, pe3, po3):
    @functools.partial(
        pl.kernel,
        out_type=jax.ShapeDtypeStruct((N, D), jnp.float32),
        mesh=_sc_mesh(),
        scratch_types=[
            pltpu.VMEM((CH, D), jnp.float32),
            pltpu.VMEM((CH, D), jnp.float32),
            pltpu.VMEM((NCH, CH), jnp.int32),
            pltpu.VMEM((NCH, CH), jnp.int32),
            pltpu.SemaphoreType.DMA,
        ],
    )
    def k(xout_hbm, pe_hbm, po_hbm, y_hbm, buf0, buf1, idxe, idxo, sem):
        w = lax.axis_index("s") * _NC + lax.axis_index("c")
        base = w * (N // _NW)
        pltpu.sync_copy(pe_hbm.at[w], idxe)
        pltpu.sync_copy(po_hbm.at[w], idxo)
        for j in range(NCH):
            pltpu.async_copy(xout_hbm.at[idxe.at[j]], buf0, sem).wait()
            pltpu.async_copy(xout_hbm.at[idxo.at[j]], buf1, sem).wait()

            def rowadd(r, carry):
                for c in range(D // 16):
                    sl = pl.ds(c * 16, 16)
                    buf0[r, sl] = buf0[r, sl] + buf1[r, sl]
                return carry

            lax.fori_loop(0, CH, rowadd, 0)
            pltpu.sync_copy(buf0, y_hbm.at[pl.ds(base + j * CH, CH)])

    return k(xout, pe3, po3)


def kernel(inputBatch, Wg, W1, W2):
    xf = inputBatch.reshape(-1, D)
    xg0, xg1, ii = _router(xf, Wg)

    ids3 = ii.reshape(GR, 1, GC)
    pos3, hist2 = _sortpos(ids3)
    pos = pos3.reshape(-1)
    hist = hist2[:, 0]

    offsets = jnp.concatenate(
        [jnp.zeros((1,), jnp.int32), jnp.cumsum(hist, dtype=jnp.int32)])

    tt = jnp.arange(NT, dtype=jnp.int32)
    es = jnp.searchsorted(offsets, tt * TM, side="right").astype(jnp.int32) - 1
    ee = jnp.searchsorted(offsets, tt * TM + (TM - 1),
                          side="right").astype(jnp.int32) - 1
    ne = ee - es + 1
    start_g = jnp.concatenate(
        [jnp.zeros((1,), jnp.int32), jnp.cumsum(ne, dtype=jnp.int32)])
    total = start_g[-1]
    gg = jnp.arange(G, dtype=jnp.int32)
    tg = jnp.clip(
        jnp.searchsorted(start_g, gg, side="right").astype(jnp.int32) - 1,
        0, NT - 1)
    eg = es[tg] + (gg - start_g[tg])
    valid = gg < total
    first = jnp.where(valid, (gg == start_g[tg]), False).astype(jnp.int32)
    tg = jnp.where(valid, tg, NT - 1)
    eg = jnp.where(valid, jnp.clip(eg, 0, E - 1), E - 1)
    valid = valid.astype(jnp.int32)

    pe3 = pos[0::2].reshape(_NW, NCH, CH)
    po3 = pos[1::2].reshape(_NW, NCH, CH)

    Xs = _scatter(xg0, xg1, pe3, po3)
    Xout = _ffn(tg, eg, first, valid, offsets, Xs, W1, W2)
    y = _combine(Xout, pe3, po3)
    return y.reshape(B, S, D)
